# baseline (speedup 1.0000x reference)
import jax
import jax.numpy as jnp
from jax import lax
from jax.experimental import pallas as pl
from jax.experimental.pallas import tpu as pltpu
from jax.experimental.pallas import tpu_sc as plsc

N = 100000
D = 512
B = 50000
NC, NS = 2, 16
NW = NC * NS
CB = 1568
BPAD = NW * CB
NPAD = 100352
SH = NPAD // NW
NV = CB // 16
GCH = 56
GT = CB // GCH
SCAN_V = (BPAD // 16) // GT
SCH = 64
ST = CB // SCH + 1

_MESH = plsc.VectorSubcoreMesh(
    core_axis_name="c", subcore_axis_name="s", num_cores=NC, num_subcores=NS
)
_SC_PARAMS = pltpu.CompilerParams(needs_layout_passes=False)


def _wid():
    return lax.axis_index("s") * NC + lax.axis_index("c")


def _gd_body(mem_ref, ids_hbm, rows_hbm, merged_hbm,
             idsall, idsg, tab, rowbuf0, rowbuf1,
             sem_g0, sem_g1, sem_s0, sem_s1):
    w = _wid()
    base = pl.multiple_of(w * CB, CB)
    base_id = pl.multiple_of(w * SH, SH)
    lanes = jnp.arange(16, dtype=jnp.int32)

    pltpu.sync_copy(ids_hbm, idsall)

    def clamp(i, _):
        sl = pl.ds(i * 16, 16)
        idsg[sl] = jnp.minimum(idsall[pl.ds(base + i * 16, 16)], N - 1)
        return 0

    lax.fori_loop(0, NV, clamp, 0)

    def initv(i, _):
        tab[pl.ds(i * 16, 16)] = jnp.full((16,), -1, jnp.int32)
        return 0

    lax.fori_loop(0, SH // 16, initv, 0)

    def scan_strip(c):
        def step(i, _):
            ids16 = idsall[pl.ds(i * 16, 16)]
            pos = (i * 16) + lanes
            inr = (ids16 >= base_id) & (ids16 < base_id + SH)
            ridx = jnp.where(inr, ids16 - base_id, 0)
            plsc.store_scatter(tab, [ridx], pos, mask=inr)
            got = plsc.load_gather(tab, [ridx])

            @pl.when(jnp.any(inr & (got < pos)))
            def _slow():
                def fix(j, _):
                    g2 = plsc.load_gather(tab, [ridx])
                    plsc.store_scatter(tab, [ridx], pos, mask=inr & (g2 < pos))
                    return 0

                lax.fori_loop(0, 15, fix, 0)

            return 0

        lax.fori_loop(c * SCAN_V, (c + 1) * SCAN_V, step, 0)

    bufs = (rowbuf0, rowbuf1)
    gsems = (sem_g0, sem_g1)
    ssems = (sem_s0, sem_s1)

    def start_gather(c):
        p = c % 2
        return pltpu.async_copy(
            mem_ref.at[idsg.at[pl.ds(c * GCH, GCH)]], bufs[p], gsems[p])

    def start_write(c):
        p = c % 2
        return pltpu.async_copy(
            bufs[p], rows_hbm.at[pl.ds(base + c * GCH, GCH)], ssems[p])

    g = {0: start_gather(0)}
    wr = {}
    for c in range(GT):
        if c + 1 < GT:
            if c - 1 >= 0:
                wr[c - 1].wait()
            g[c + 1] = start_gather(c + 1)
        scan_strip(c)
        g[c].wait()
        wr[c] = start_write(c)
    wr[GT - 2].wait()
    wr[GT - 1].wait()

    pltpu.sync_copy(tab, merged_hbm.at[pl.ds(base_id, SH)])


_gd = pl.kernel(
    _gd_body,
    out_type=(
        jax.ShapeDtypeStruct((BPAD, D), jnp.float32),
        jax.ShapeDtypeStruct((NPAD,), jnp.int32),
    ),
    mesh=_MESH,
    compiler_params=_SC_PARAMS,
    scratch_types=[
        pltpu.VMEM((BPAD,), jnp.int32),
        pltpu.VMEM((CB,), jnp.int32),
        pltpu.VMEM((SH,), jnp.int32),
        pltpu.VMEM((GCH, D), jnp.float32),
        pltpu.VMEM((GCH, D), jnp.float32),
        pltpu.SemaphoreType.DMA,
        pltpu.SemaphoreType.DMA,
        pltpu.SemaphoreType.DMA,
        pltpu.SemaphoreType.DMA,
    ],
)


def _scatter_body(mem_ref, ids_hbm, merged_hbm, newrows_hbm,
                  idsbuf, mvals, plist, ilist, pch0, ich0, pch1, ich1,
                  rowbuf0, rowbuf1,
                  sem_e, sem_g0, sem_g1, sem_s0, sem_s1):
    base = pl.multiple_of(_wid() * CB, CB)
    lanes = jnp.arange(16, dtype=jnp.int32)
    pltpu.sync_copy(ids_hbm.at[pl.ds(base, CB)], idsbuf)

    EG = 112
    for c in range(CB // EG):
        sl = pl.ds(c * EG, EG)
        pltpu.async_copy(merged_hbm.at[idsbuf.at[sl]], mvals.at[sl], sem_e).wait()

    def comp(i, carry):
        cnt, p0, i0 = carry
        sl = pl.ds(i * 16, 16)
        ids16 = idsbuf[sl]
        pos = (base + i * 16) + lanes
        m = (mvals[sl] == pos) & (pos < B)
        plsc.store_compressed(plist.at[pl.ds(cnt, 16)], pos, mask=m)
        plsc.store_compressed(ilist.at[pl.ds(cnt, 16)], ids16, mask=m)
        pmin = jnp.min(jnp.where(m, pos, jnp.int32(2 ** 30)))
        imin = jnp.max(jnp.where(m & (pos == pmin), ids16, -1))
        first = (cnt == 0) & jnp.any(m)
        p0 = jnp.where(first, pmin, p0)
        i0 = jnp.where(first, imin, i0)
        return cnt + jnp.sum(m.astype(jnp.int32)), p0, i0

    cnt, p0, i0 = lax.fori_loop(
        0, NV, comp, (jnp.int32(0), jnp.int32(0), jnp.int32(0)))
    nch = (cnt + (SCH - 1)) // SCH

    @pl.when(cnt > 0)
    def _san():
        def san(i, _):
            sl = pl.ds(i * 16, 16)
            g = i * 16 + lanes
            valid = g < cnt
            ilist[sl] = jnp.where(valid, ilist[sl], i0)
            plist[sl] = jnp.where(valid, plist[sl], p0)
            return 0

        lax.fori_loop(0, (CB + SCH) // 16, san, 0)

    pchs = (pch0, pch1)
    ichs = (ich0, ich1)
    bufs = (rowbuf0, rowbuf1)
    gsems = (sem_g0, sem_g1)
    ssems = (sem_s0, sem_s1)

    for k in range(ST):
        p = k % 2

        @pl.when(k < nch)
        def _do(k=k, p=p):
            if k >= 2:
                pltpu.make_async_copy(bufs[p], mem_ref.at[ichs[p]], ssems[p]).wait()
            st = k * SCH
            for j in range(SCH // 16):
                pchs[p][pl.ds(j * 16, 16)] = plist[pl.ds(st + j * 16, 16)]
                ichs[p][pl.ds(j * 16, 16)] = ilist[pl.ds(st + j * 16, 16)]
            pltpu.async_copy(newrows_hbm.at[pchs[p]], bufs[p], gsems[p]).wait()
            pltpu.async_copy(bufs[p], mem_ref.at[ichs[p]], ssems[p])

    for p in range(2):
        @pl.when(nch > p)
        def _drain(p=p):
            pltpu.make_async_copy(bufs[p], mem_ref.at[ichs[p]], ssems[p]).wait()


_scatter = pl.kernel(
    _scatter_body,
    out_type=(),
    mesh=_MESH,
    compiler_params=_SC_PARAMS,
    scratch_types=[
        pltpu.VMEM((CB,), jnp.int32),
        pltpu.VMEM((CB,), jnp.int32),
        pltpu.VMEM((CB + SCH,), jnp.int32),
        pltpu.VMEM((CB + SCH,), jnp.int32),
        pltpu.VMEM((SCH,), jnp.int32),
        pltpu.VMEM((SCH,), jnp.int32),
        pltpu.VMEM((SCH,), jnp.int32),
        pltpu.VMEM((SCH,), jnp.int32),
        pltpu.VMEM((SCH, D), jnp.float32),
        pltpu.VMEM((SCH, D), jnp.float32),
        pltpu.SemaphoreType.DMA,
        pltpu.SemaphoreType.DMA,
        pltpu.SemaphoreType.DMA,
        pltpu.SemaphoreType.DMA,
        pltpu.SemaphoreType.DMA,
    ],
)


BM = 2000


def _tc_body(msg_ref, rows_ref, wc_ref, wh_ref, b_ref, out_ref):
    msg = msg_ref[...]
    rows = rows_ref[...]
    dn = (((1,), (1,)), ((), ()))
    i_c = lax.dot_general(msg, wc_ref[...], dn, preferred_element_type=jnp.float32)
    h = lax.dot_general(rows, wh_ref[...], dn, preferred_element_type=jnp.float32)
    s = i_c + h + b_ref[...]
    gate = jax.nn.sigmoid(s[:, :D])
    h_c = jnp.tanh(s[:, D:])
    out_ref[...] = (1.0 - gate) * h_c + gate * rows


def _tc_update(msg, rows_pad, wc, wh, bias2d):
    return pl.pallas_call(
        _tc_body,
        grid=(B // BM,),
        in_specs=[
            pl.BlockSpec((BM, D), lambda i: (i, 0)),
            pl.BlockSpec((BM, D), lambda i: (i, 0)),
            pl.BlockSpec((2 * D, D), lambda i: (0, 0)),
            pl.BlockSpec((2 * D, D), lambda i: (0, 0)),
            pl.BlockSpec((1, 2 * D), lambda i: (0, 0)),
        ],
        out_specs=pl.BlockSpec((BM, D), lambda i: (i, 0)),
        out_shape=jax.ShapeDtypeStruct((B, D), jnp.float32),
    )(msg, rows_pad, wc, wh, bias2d)


def kernel(unique_node_ids_list, unique_messages_list, mem, last_update,
           W_C_w, W_C_b, W_h_w, W_h_b):
    ids_l = unique_node_ids_list.astype(jnp.int32)
    bias2d = (W_C_b + W_h_b).reshape(1, 2 * D)
    ref = jax.new_ref(mem)
    padids = N + jnp.arange(BPAD - B, dtype=jnp.int32)
    divide = ids_l.shape[0]
    for k in range(divide):
        idsp = jnp.concatenate([ids_l[k], padids])
        rows_pad, merged = _gd(ref, idsp)
        newrows = _tc_update(unique_messages_list[k], rows_pad, W_C_w, W_h_w, bias2d)
        _scatter(ref, idsp, merged, newrows)
    out = ref[...]
    return out, last_update


# speedup vs baseline: 1.5423x; 1.5423x over previous
import jax
import jax.numpy as jnp
from jax import lax
from jax.experimental import pallas as pl
from jax.experimental.pallas import tpu as pltpu
from jax.experimental.pallas import tpu_sc as plsc

N = 100000
D = 512
B = 50000
NC, NS = 2, 16
NW = NC * NS
CB = 1568
BPAD = NW * CB
NPAD = 100352
SH = NPAD // NW
NV = CB // 16
GCH = 56
GT = CB // GCH
SCAN_V = (BPAD // 16) // GT
SCH = 64
ST = CB // SCH + 1
DIV = 4

_MESH = plsc.VectorSubcoreMesh(
    core_axis_name="c", subcore_axis_name="s", num_cores=NC, num_subcores=NS
)
_SC_PARAMS = pltpu.CompilerParams(needs_layout_passes=False)


def _wid():
    return lax.axis_index("s") * NC + lax.axis_index("c")


def _g_body(mem_ref, ids_hbm, rows_hbm, idsbuf, idsg, rowbuf0, rowbuf1,
            sem_g0, sem_g1, sem_s0, sem_s1):
    base = pl.multiple_of(_wid() * CB, CB)
    pltpu.sync_copy(ids_hbm.at[pl.ds(base, CB)], idsbuf)

    def clamp(i, _):
        sl = pl.ds(i * 16, 16)
        idsg[sl] = jnp.minimum(idsbuf[sl], N - 1)
        return 0

    lax.fori_loop(0, NV, clamp, 0)

    bufs = (rowbuf0, rowbuf1)
    gsems = (sem_g0, sem_g1)
    ssems = (sem_s0, sem_s1)

    def start_gather(c):
        p = c % 2
        return pltpu.async_copy(
            mem_ref.at[idsg.at[pl.ds(c * GCH, GCH)]], bufs[p], gsems[p])

    def start_write(c):
        p = c % 2
        return pltpu.async_copy(
            bufs[p], rows_hbm.at[pl.ds(base + c * GCH, GCH)], ssems[p])

    g = {0: start_gather(0)}
    wr = {}
    for c in range(GT):
        if c + 1 < GT:
            if c - 1 >= 0:
                wr[c - 1].wait()
            g[c + 1] = start_gather(c + 1)
        g[c].wait()
        wr[c] = start_write(c)
    wr[GT - 2].wait()
    wr[GT - 1].wait()


_g = pl.kernel(
    _g_body,
    out_type=jax.ShapeDtypeStruct((BPAD, D), jnp.float32),
    mesh=_MESH,
    compiler_params=_SC_PARAMS,
    scratch_types=[
        pltpu.VMEM((CB,), jnp.int32),
        pltpu.VMEM((CB,), jnp.int32),
        pltpu.VMEM((GCH, D), jnp.float32),
        pltpu.VMEM((GCH, D), jnp.float32),
        pltpu.SemaphoreType.DMA,
        pltpu.SemaphoreType.DMA,
        pltpu.SemaphoreType.DMA,
        pltpu.SemaphoreType.DMA,
    ],
)


def _a1_body(ids4_hbm, tabs4_hbm, idsbuf, tab):
    w = _wid()
    base = pl.multiple_of(w * CB, CB)
    lanes = jnp.arange(16, dtype=jnp.int32)

    for blk in range(DIV):
        def initv(i, _):
            tab[pl.ds(i * 16, 16)] = jnp.full((16,), -1, jnp.int32)
            return 0

        lax.fori_loop(0, NPAD // 16, initv, 0)
        pltpu.sync_copy(ids4_hbm.at[pl.ds(blk * BPAD + base, CB)], idsbuf)

        def step(i, _):
            ids16 = idsbuf[pl.ds(i * 16, 16)]
            pos = (base + i * 16) + lanes
            plsc.store_scatter(tab, [ids16], pos, mask=jnp.full((16,), True))
            got = plsc.load_gather(tab, [ids16])

            @pl.when(jnp.any(got < pos))
            def _slow():
                def fix(j, _):
                    g2 = plsc.load_gather(tab, [ids16])
                    plsc.store_scatter(tab, [ids16], pos, mask=g2 < pos)
                    return 0

                lax.fori_loop(0, 15, fix, 0)

            return 0

        lax.fori_loop(0, NV, step, 0)
        pltpu.sync_copy(
            tab, tabs4_hbm.at[pl.ds(pl.multiple_of((blk * NW + w) * NPAD, NPAD), NPAD)])


_a1 = pl.kernel(
    _a1_body,
    out_type=jax.ShapeDtypeStruct((DIV * NW * NPAD,), jnp.int32),
    mesh=_MESH,
    compiler_params=_SC_PARAMS,
    scratch_types=[
        pltpu.VMEM((CB,), jnp.int32),
        pltpu.VMEM((NPAD,), jnp.int32),
    ],
)


def _a2_body(tabs4_hbm, merged4_hbm, buf2, acc, sem):
    off = pl.multiple_of(_wid() * SH, SH)

    for blk in range(DIV):
        for w0 in range(0, NW, 8):
            descs = [
                pltpu.async_copy(
                    tabs4_hbm.at[pl.ds((blk * NW + t) * NPAD + off, SH)],
                    buf2.at[pl.ds(t * SH, SH)], sem)
                for t in range(w0, w0 + 8)
            ]
            for d in descs:
                d.wait()

        def step(v, _):
            m = buf2[pl.ds(v * 16, 16)]
            for t in range(1, NW):
                m = jnp.maximum(m, buf2[pl.ds(t * SH + v * 16, 16)])
            acc[pl.ds(v * 16, 16)] = m
            return 0

        lax.fori_loop(0, SH // 16, step, 0)
        pltpu.sync_copy(acc, merged4_hbm.at[pl.ds(blk * NPAD + off, SH)])


_a2 = pl.kernel(
    _a2_body,
    out_type=jax.ShapeDtypeStruct((DIV * NPAD,), jnp.int32),
    mesh=_MESH,
    compiler_params=_SC_PARAMS,
    scratch_types=[
        pltpu.VMEM((NW * SH,), jnp.int32),
        pltpu.VMEM((SH,), jnp.int32),
        pltpu.SemaphoreType.DMA,
    ],
)


def _scatter_body(mem_ref, ids_hbm, merged_hbm, newrows_hbm,
                  idsbuf, mvals, plist, ilist, pch0, ich0, pch1, ich1,
                  rowbuf0, rowbuf1,
                  sem_e, sem_g0, sem_g1, sem_s0, sem_s1):
    base = pl.multiple_of(_wid() * CB, CB)
    lanes = jnp.arange(16, dtype=jnp.int32)
    pltpu.sync_copy(ids_hbm.at[pl.ds(base, CB)], idsbuf)

    EG = 112
    for c in range(CB // EG):
        sl = pl.ds(c * EG, EG)
        pltpu.async_copy(merged_hbm.at[idsbuf.at[sl]], mvals.at[sl], sem_e).wait()

    def comp(i, carry):
        cnt, p0, i0 = carry
        sl = pl.ds(i * 16, 16)
        ids16 = idsbuf[sl]
        pos = (base + i * 16) + lanes
        m = (mvals[sl] == pos) & (pos < B)
        plsc.store_compressed(plist.at[pl.ds(cnt, 16)], pos, mask=m)
        plsc.store_compressed(ilist.at[pl.ds(cnt, 16)], ids16, mask=m)
        pmin = jnp.min(jnp.where(m, pos, jnp.int32(2 ** 30)))
        imin = jnp.max(jnp.where(m & (pos == pmin), ids16, -1))
        first = (cnt == 0) & jnp.any(m)
        p0 = jnp.where(first, pmin, p0)
        i0 = jnp.where(first, imin, i0)
        return cnt + jnp.sum(m.astype(jnp.int32)), p0, i0

    cnt, p0, i0 = lax.fori_loop(
        0, NV, comp, (jnp.int32(0), jnp.int32(0), jnp.int32(0)))
    nch = (cnt + (SCH - 1)) // SCH

    @pl.when(cnt > 0)
    def _san():
        def san(i, _):
            sl = pl.ds(i * 16, 16)
            g = i * 16 + lanes
            valid = g < cnt
            ilist[sl] = jnp.where(valid, ilist[sl], i0)
            plist[sl] = jnp.where(valid, plist[sl], p0)
            return 0

        lax.fori_loop(0, (CB + SCH) // 16, san, 0)

    pchs = (pch0, pch1)
    ichs = (ich0, ich1)
    bufs = (rowbuf0, rowbuf1)
    gsems = (sem_g0, sem_g1)
    ssems = (sem_s0, sem_s1)

    for k in range(ST):
        p = k % 2

        @pl.when(k < nch)
        def _do(k=k, p=p):
            if k >= 2:
                pltpu.make_async_copy(bufs[p], mem_ref.at[ichs[p]], ssems[p]).wait()
            st = k * SCH
            for j in range(SCH // 16):
                pchs[p][pl.ds(j * 16, 16)] = plist[pl.ds(st + j * 16, 16)]
                ichs[p][pl.ds(j * 16, 16)] = ilist[pl.ds(st + j * 16, 16)]
            pltpu.async_copy(newrows_hbm.at[pchs[p]], bufs[p], gsems[p]).wait()
            pltpu.async_copy(bufs[p], mem_ref.at[ichs[p]], ssems[p])

    for p in range(2):
        @pl.when(nch > p)
        def _drain(p=p):
            pltpu.make_async_copy(bufs[p], mem_ref.at[ichs[p]], ssems[p]).wait()


_scatter = pl.kernel(
    _scatter_body,
    out_type=(),
    mesh=_MESH,
    compiler_params=_SC_PARAMS,
    scratch_types=[
        pltpu.VMEM((CB,), jnp.int32),
        pltpu.VMEM((CB,), jnp.int32),
        pltpu.VMEM((CB + SCH,), jnp.int32),
        pltpu.VMEM((CB + SCH,), jnp.int32),
        pltpu.VMEM((SCH,), jnp.int32),
        pltpu.VMEM((SCH,), jnp.int32),
        pltpu.VMEM((SCH,), jnp.int32),
        pltpu.VMEM((SCH,), jnp.int32),
        pltpu.VMEM((SCH, D), jnp.float32),
        pltpu.VMEM((SCH, D), jnp.float32),
        pltpu.SemaphoreType.DMA,
        pltpu.SemaphoreType.DMA,
        pltpu.SemaphoreType.DMA,
        pltpu.SemaphoreType.DMA,
        pltpu.SemaphoreType.DMA,
    ],
)


BM = 2000


def _tc_body(msg_ref, rows_ref, wc_ref, wh_ref, b_ref, out_ref):
    msg = msg_ref[...]
    rows = rows_ref[...]
    dn = (((1,), (1,)), ((), ()))
    i_c = lax.dot_general(msg, wc_ref[...], dn, preferred_element_type=jnp.float32)
    h = lax.dot_general(rows, wh_ref[...], dn, preferred_element_type=jnp.float32)
    s = i_c + h + b_ref[...]
    gate = jax.nn.sigmoid(s[:, :D])
    h_c = jnp.tanh(s[:, D:])
    out_ref[...] = (1.0 - gate) * h_c + gate * rows


def _tc_update(msg, rows_pad, wc, wh, bias2d):
    return pl.pallas_call(
        _tc_body,
        grid=(B // BM,),
        in_specs=[
            pl.BlockSpec((BM, D), lambda i: (i, 0)),
            pl.BlockSpec((BM, D), lambda i: (i, 0)),
            pl.BlockSpec((2 * D, D), lambda i: (0, 0)),
            pl.BlockSpec((2 * D, D), lambda i: (0, 0)),
            pl.BlockSpec((1, 2 * D), lambda i: (0, 0)),
        ],
        out_specs=pl.BlockSpec((BM, D), lambda i: (i, 0)),
        out_shape=jax.ShapeDtypeStruct((B, D), jnp.float32),
    )(msg, rows_pad, wc, wh, bias2d)


def kernel(unique_node_ids_list, unique_messages_list, mem, last_update,
           W_C_w, W_C_b, W_h_w, W_h_b):
    ids_l = unique_node_ids_list.astype(jnp.int32)
    bias2d = (W_C_b + W_h_b).reshape(1, 2 * D)
    ref = jax.new_ref(mem)
    padids = N + jnp.arange(BPAD - B, dtype=jnp.int32)
    divide = ids_l.shape[0]
    idsp = [jnp.concatenate([ids_l[k], padids]) for k in range(divide)]
    ids4 = jnp.concatenate(idsp)
    tabs4 = _a1(ids4)
    merged4 = _a2(tabs4)
    for k in range(divide):
        rows_pad = _g(ref, idsp[k])
        newrows = _tc_update(unique_messages_list[k], rows_pad, W_C_w, W_h_w, bias2d)
        _scatter(ref, idsp[k], merged4[k * NPAD:(k + 1) * NPAD], newrows)
    out = ref[...]
    return out, last_update


# speedup vs baseline: 1.6289x; 1.0561x over previous
import jax
import jax.numpy as jnp
from jax import lax
from jax.experimental import pallas as pl
from jax.experimental.pallas import tpu as pltpu
from jax.experimental.pallas import tpu_sc as plsc

N = 100000
D = 512
B = 50000
NC, NS = 2, 16
NW = NC * NS
CB = 1568
BPAD = NW * CB
NPAD = 102400
SH = NPAD // NW
NV = CB // 16
GCH = 56
GT = CB // GCH
SCAN_V = (BPAD // 16) // GT
SCH = 64
ST = CB // SCH + 1
DIV = 4

_MESH = plsc.VectorSubcoreMesh(
    core_axis_name="c", subcore_axis_name="s", num_cores=NC, num_subcores=NS
)
_SC_PARAMS = pltpu.CompilerParams(needs_layout_passes=False)


def _wid():
    return lax.axis_index("s") * NC + lax.axis_index("c")


def _g_body(mem_ref, ids_hbm, rows_hbm, idsbuf, idsg, rowbuf0, rowbuf1,
            sem_g0, sem_g1, sem_s0, sem_s1):
    base = pl.multiple_of(_wid() * CB, CB)
    pltpu.sync_copy(ids_hbm.at[pl.ds(base, CB)], idsbuf)

    def clamp(i, _):
        sl = pl.ds(i * 16, 16)
        idsg[sl] = jnp.minimum(idsbuf[sl], N - 1)
        return 0

    lax.fori_loop(0, NV, clamp, 0)

    bufs = (rowbuf0, rowbuf1)
    gsems = (sem_g0, sem_g1)
    ssems = (sem_s0, sem_s1)

    def start_gather(c):
        p = c % 2
        return pltpu.async_copy(
            mem_ref.at[idsg.at[pl.ds(c * GCH, GCH)]], bufs[p], gsems[p])

    def start_write(c):
        p = c % 2
        return pltpu.async_copy(
            bufs[p], rows_hbm.at[pl.ds(base + c * GCH, GCH)], ssems[p])

    g = {0: start_gather(0)}
    wr = {}
    for c in range(GT):
        if c + 1 < GT:
            if c - 1 >= 0:
                wr[c - 1].wait()
            g[c + 1] = start_gather(c + 1)
        g[c].wait()
        wr[c] = start_write(c)
    wr[GT - 2].wait()
    wr[GT - 1].wait()


_g = pl.kernel(
    _g_body,
    out_type=jax.ShapeDtypeStruct((BPAD, D), jnp.float32),
    mesh=_MESH,
    compiler_params=_SC_PARAMS,
    scratch_types=[
        pltpu.VMEM((CB,), jnp.int32),
        pltpu.VMEM((CB,), jnp.int32),
        pltpu.VMEM((GCH, D), jnp.float32),
        pltpu.VMEM((GCH, D), jnp.float32),
        pltpu.SemaphoreType.DMA,
        pltpu.SemaphoreType.DMA,
        pltpu.SemaphoreType.DMA,
        pltpu.SemaphoreType.DMA,
    ],
)


def _a1_body(ids4_hbm, tabs4_hbm, idsbuf, tab):
    w = _wid()
    base = pl.multiple_of(w * CB, CB)
    lanes = jnp.arange(16, dtype=jnp.int32)

    for blk in range(DIV):
        neg1 = jnp.full((16,), -1, jnp.int32)

        def initv(i, _):
            for u in range(8):
                tab[pl.ds(i * 128 + u * 16, 16)] = neg1
            return 0

        lax.fori_loop(0, NPAD // 128, initv, 0)
        pltpu.sync_copy(ids4_hbm.at[pl.ds(blk * BPAD + base, CB)], idsbuf)

        def step(i, _):
            ids16 = idsbuf[pl.ds(i * 16, 16)]
            pos = (base + i * 16) + lanes
            plsc.store_scatter(tab, [ids16], pos, mask=jnp.full((16,), True))
            got = plsc.load_gather(tab, [ids16])

            @pl.when(jnp.any(got < pos))
            def _slow():
                def fix(j, _):
                    g2 = plsc.load_gather(tab, [ids16])
                    plsc.store_scatter(tab, [ids16], pos, mask=g2 < pos)
                    return 0

                lax.fori_loop(0, 15, fix, 0)

            return 0

        lax.fori_loop(0, NV, step, 0)
        pltpu.sync_copy(tab, tabs4_hbm.at[blk * NW + w])


_a1 = pl.kernel(
    _a1_body,
    out_type=jax.ShapeDtypeStruct((DIV * NW, NPAD), jnp.int32),
    mesh=_MESH,
    compiler_params=_SC_PARAMS,
    scratch_types=[
        pltpu.VMEM((CB,), jnp.int32),
        pltpu.VMEM((NPAD,), jnp.int32),
    ],
)


def _a2_body(tabs4_hbm, merged4_hbm, buf2, acc, sem):
    off = pl.multiple_of(_wid() * SH, SH)

    for blk in range(DIV):
        pltpu.async_copy(
            tabs4_hbm.at[pl.ds(blk * NW, NW), pl.ds(off, SH)], buf2, sem).wait()

        def step(v, _):
            m = buf2[0, pl.ds(v * 16, 16)]
            for t in range(1, NW):
                m = jnp.maximum(m, buf2[t, pl.ds(v * 16, 16)])
            acc[pl.ds(v * 16, 16)] = m
            return 0

        lax.fori_loop(0, SH // 16, step, 0)
        pltpu.sync_copy(acc, merged4_hbm.at[pl.ds(blk * NPAD + off, SH)])


_a2 = pl.kernel(
    _a2_body,
    out_type=jax.ShapeDtypeStruct((DIV * NPAD,), jnp.int32),
    mesh=_MESH,
    compiler_params=_SC_PARAMS,
    scratch_types=[
        pltpu.VMEM((NW, SH), jnp.int32),
        pltpu.VMEM((SH,), jnp.int32),
        pltpu.SemaphoreType.DMA,
    ],
)


def _scatter_body(mem_ref, ids_hbm, merged_hbm, newrows_hbm,
                  idsbuf, mvals, plist, ilist, pch0, ich0, pch1, ich1,
                  rowbuf0, rowbuf1,
                  sem_e, sem_g0, sem_g1, sem_s0, sem_s1):
    base = pl.multiple_of(_wid() * CB, CB)
    lanes = jnp.arange(16, dtype=jnp.int32)
    pltpu.sync_copy(ids_hbm.at[pl.ds(base, CB)], idsbuf)

    EG = 112
    for c in range(CB // EG):
        sl = pl.ds(c * EG, EG)
        pltpu.async_copy(merged_hbm.at[idsbuf.at[sl]], mvals.at[sl], sem_e).wait()

    def comp(i, carry):
        cnt, p0, i0 = carry
        sl = pl.ds(i * 16, 16)
        ids16 = idsbuf[sl]
        pos = (base + i * 16) + lanes
        m = (mvals[sl] == pos) & (pos < B)
        plsc.store_compressed(plist.at[pl.ds(cnt, 16)], pos, mask=m)
        plsc.store_compressed(ilist.at[pl.ds(cnt, 16)], ids16, mask=m)
        pmin = jnp.min(jnp.where(m, pos, jnp.int32(2 ** 30)))
        imin = jnp.max(jnp.where(m & (pos == pmin), ids16, -1))
        first = (cnt == 0) & jnp.any(m)
        p0 = jnp.where(first, pmin, p0)
        i0 = jnp.where(first, imin, i0)
        return cnt + jnp.sum(m.astype(jnp.int32)), p0, i0

    cnt, p0, i0 = lax.fori_loop(
        0, NV, comp, (jnp.int32(0), jnp.int32(0), jnp.int32(0)))
    nch = (cnt + (SCH - 1)) // SCH

    @pl.when(cnt > 0)
    def _san():
        def san(i, _):
            sl = pl.ds(i * 16, 16)
            g = i * 16 + lanes
            valid = g < cnt
            ilist[sl] = jnp.where(valid, ilist[sl], i0)
            plist[sl] = jnp.where(valid, plist[sl], p0)
            return 0

        lax.fori_loop(0, (CB + SCH) // 16, san, 0)

    pchs = (pch0, pch1)
    ichs = (ich0, ich1)
    bufs = (rowbuf0, rowbuf1)
    gsems = (sem_g0, sem_g1)
    ssems = (sem_s0, sem_s1)

    for k in range(ST):
        p = k % 2

        @pl.when(k < nch)
        def _do(k=k, p=p):
            if k >= 2:
                pltpu.make_async_copy(bufs[p], mem_ref.at[ichs[p]], ssems[p]).wait()
            st = k * SCH
            for j in range(SCH // 16):
                pchs[p][pl.ds(j * 16, 16)] = plist[pl.ds(st + j * 16, 16)]
                ichs[p][pl.ds(j * 16, 16)] = ilist[pl.ds(st + j * 16, 16)]
            pltpu.async_copy(newrows_hbm.at[pchs[p]], bufs[p], gsems[p]).wait()
            pltpu.async_copy(bufs[p], mem_ref.at[ichs[p]], ssems[p])

    for p in range(2):
        @pl.when(nch > p)
        def _drain(p=p):
            pltpu.make_async_copy(bufs[p], mem_ref.at[ichs[p]], ssems[p]).wait()


_scatter = pl.kernel(
    _scatter_body,
    out_type=(),
    mesh=_MESH,
    compiler_params=_SC_PARAMS,
    scratch_types=[
        pltpu.VMEM((CB,), jnp.int32),
        pltpu.VMEM((CB,), jnp.int32),
        pltpu.VMEM((CB + SCH,), jnp.int32),
        pltpu.VMEM((CB + SCH,), jnp.int32),
        pltpu.VMEM((SCH,), jnp.int32),
        pltpu.VMEM((SCH,), jnp.int32),
        pltpu.VMEM((SCH,), jnp.int32),
        pltpu.VMEM((SCH,), jnp.int32),
        pltpu.VMEM((SCH, D), jnp.float32),
        pltpu.VMEM((SCH, D), jnp.float32),
        pltpu.SemaphoreType.DMA,
        pltpu.SemaphoreType.DMA,
        pltpu.SemaphoreType.DMA,
        pltpu.SemaphoreType.DMA,
        pltpu.SemaphoreType.DMA,
    ],
)


BM = 2000


def _tc_body(msg_ref, rows_ref, wc_ref, wh_ref, b_ref, out_ref):
    msg = msg_ref[...].astype(jnp.bfloat16)
    rows = rows_ref[...]
    dn = (((1,), (1,)), ((), ()))
    i_c = lax.dot_general(msg, wc_ref[...], dn, preferred_element_type=jnp.float32)
    h = lax.dot_general(rows.astype(jnp.bfloat16), wh_ref[...], dn,
                        preferred_element_type=jnp.float32)
    s = i_c + h + b_ref[...]
    gate = jax.nn.sigmoid(s[:, :D])
    h_c = jnp.tanh(s[:, D:])
    out_ref[...] = (1.0 - gate) * h_c + gate * rows


def _tc_update(msg, rows_pad, wc, wh, bias2d):
    return pl.pallas_call(
        _tc_body,
        grid=(B // BM,),
        in_specs=[
            pl.BlockSpec((BM, D), lambda i: (i, 0)),
            pl.BlockSpec((BM, D), lambda i: (i, 0)),
            pl.BlockSpec((2 * D, D), lambda i: (0, 0)),
            pl.BlockSpec((2 * D, D), lambda i: (0, 0)),
            pl.BlockSpec((1, 2 * D), lambda i: (0, 0)),
        ],
        out_specs=pl.BlockSpec((BM, D), lambda i: (i, 0)),
        out_shape=jax.ShapeDtypeStruct((B, D), jnp.float32),
    )(msg, rows_pad, wc, wh, bias2d)


def kernel(unique_node_ids_list, unique_messages_list, mem, last_update,
           W_C_w, W_C_b, W_h_w, W_h_b):
    ids_l = unique_node_ids_list.astype(jnp.int32)
    bias2d = (W_C_b + W_h_b).reshape(1, 2 * D)
    ref = jax.new_ref(mem)
    padids = N + jnp.arange(BPAD - B, dtype=jnp.int32)
    divide = ids_l.shape[0]
    idsp = [jnp.concatenate([ids_l[k], padids]) for k in range(divide)]
    ids4 = jnp.concatenate(idsp)
    wc_bf = W_C_w.astype(jnp.bfloat16)
    wh_bf = W_h_w.astype(jnp.bfloat16)
    rows_pad = _g(ref, idsp[0])
    tabs4 = _a1(ids4)
    merged4 = _a2(tabs4)
    for k in range(divide):
        if k > 0:
            rows_pad = _g(ref, idsp[k])
        newrows = _tc_update(unique_messages_list[k], rows_pad, wc_bf, wh_bf, bias2d)
        _scatter(ref, idsp[k], merged4[k * NPAD:(k + 1) * NPAD], newrows)
    out = ref[...]
    return out, last_update
